# baseline (speedup 1.0000x reference)
import functools

import jax
import jax.numpy as jnp
from jax import lax
from jax.experimental import pallas as pl
from jax.experimental.pallas import tpu as pltpu
from jax.experimental.pallas import tpu_sc as plsc

_B = 16384
_D = 128
_C = 1000
_NC = 2
_NS = 16
_NW = _NC * _NS
_RPW = _B // _NW
_CH = 128
_NCH = _RPW // _CH
_NBUF = 3
_LANES = 16
_JV = _D // _LANES
_DW = _D // 2
_CPT = 64


def _sc_body(latent_hbm, labels_hbm, centers_hbm, out_hbm, lab_v,
             res_v, cen_sh, out_sem, stage_sem, *bufs_and_sems):
    lat_bufs = bufs_and_sems[:_NBUF]
    cen_bufs = bufs_and_sems[_NBUF:2 * _NBUF]
    fill_sems = bufs_and_sems[2 * _NBUF:3 * _NBUF]
    gat_sems = bufs_and_sems[3 * _NBUF:]
    sid = lax.axis_index("s")
    wid = sid * _NC + lax.axis_index("c")

    def fill(ch):
        b = ch % _NBUF
        row0 = wid * _RPW + ch * _CH
        return pltpu.async_copy(
            latent_hbm.at[pl.ds(row0, _CH)], lat_bufs[b], fill_sems[b])

    def gather(ch):
        b = ch % _NBUF
        return pltpu.async_copy(
            cen_sh.at[lab_v.at[pl.ds(ch * _CH, _CH)]], cen_bufs[b],
            gat_sems[b])

    fills = {ch: fill(ch) for ch in range(_NBUF)}

    base = jnp.minimum(sid * _CPT, _C - _CPT)
    lab_copy = pltpu.async_copy(
        labels_hbm.at[pl.ds(wid * _RPW, _RPW)], lab_v, stage_sem)
    stg_copy = pltpu.async_copy(
        centers_hbm.at[pl.ds(base, _CPT)], cen_sh.at[pl.ds(base, _CPT)],
        stage_sem)

    accs = tuple(jnp.zeros((_LANES,), jnp.float32) for _ in range(_JV))

    lab_copy.wait()
    stg_copy.wait()
    plsc.subcore_barrier()
    gats = {ch: gather(ch) for ch in range(_NBUF)}
    for ch in range(_NCH):
        b = ch % _NBUF
        fills.pop(ch).wait()
        gats.pop(ch).wait()
        lat_v, cen_v = lat_bufs[b], cen_bufs[b]

        @plsc.parallel_loop(0, _CH, 1, unroll=4, carry=accs)
        def row_loop(r, acc_in):
            new = list(acc_in)
            for j in range(_JV // 2):
                wl = lat_v[r, pl.ds(j * _LANES, _LANES)]
                wc = cen_v[r, pl.ds(j * _LANES, _LANES)]
                lt = plsc.bitcast(wl, jnp.bfloat16)
                cn = plsc.bitcast(wc, jnp.bfloat16)
                a, b2 = plsc.unpack(
                    lt - cn, format=plsc.PackFormat.INTERLEAVED)
                k = 2 * j
                new[k] = new[k] + a * a
                new[k + 1] = new[k + 1] + b2 * b2
            return tuple(new)

        accs = row_loop
        if ch + _NBUF < _NCH:
            fills[ch + _NBUF] = fill(ch + _NBUF)
            gats[ch + _NBUF] = gather(ch + _NBUF)

    total = accs[0]
    for j in range(1, _JV):
        total = total + accs[j]
    res_v[...] = total
    pltpu.async_copy(res_v, out_hbm.at[wid], out_sem).wait()


@jax.jit
def _center_loss_partials(latent, labels1d, centers):
    mesh = plsc.VectorSubcoreMesh(core_axis_name="c", subcore_axis_name="s")
    run = functools.partial(
        pl.kernel,
        out_type=jax.ShapeDtypeStruct((_NW, _LANES), jnp.float32),
        mesh=mesh,
        compiler_params=pltpu.CompilerParams(needs_layout_passes=False),
        scratch_types=(
            [
                pltpu.VMEM((_RPW,), jnp.int32),
                pltpu.VMEM((_LANES,), jnp.float32),
                pltpu.VMEM_SHARED((_C, _DW), jnp.int32),
                pltpu.SemaphoreType.DMA,
                pltpu.SemaphoreType.DMA,
            ]
            + [pltpu.VMEM((_CH, _DW), jnp.int32) for _ in range(2 * _NBUF)]
            + [pltpu.SemaphoreType.DMA for _ in range(2 * _NBUF)]
        ),
    )(_sc_body)
    return run(latent, labels1d, centers)


def kernel(latent, labels, centers):
    lat32 = lax.bitcast_convert_type(
        latent.astype(jnp.bfloat16).reshape(_B, _DW, 2), jnp.int32)
    cen32 = lax.bitcast_convert_type(
        centers.astype(jnp.bfloat16).reshape(_C, _DW, 2), jnp.int32)
    partials = _center_loss_partials(lat32, labels.astype(jnp.int32), cen32)
    return jnp.sum(partials) / jnp.float32(_B)


# speedup vs baseline: 2.3010x; 2.3010x over previous
import functools

import jax
import jax.numpy as jnp
from jax import lax
from jax.experimental import pallas as pl
from jax.experimental.pallas import tpu as pltpu
from jax.experimental.pallas import tpu_sc as plsc

_B = 16384
_D = 128
_C = 1000
_NC = 2
_NS = 16
_NW = _NC * _NS
_RPW = _B // _NW
_CH = 128
_NCH = _RPW // _CH
_NBUF = 3
_LANES = 16
_JV = _D // _LANES
_CPT = 64


def _sc_body(latent_hbm, labels_hbm, centers_hbm, out_hbm, lab_v,
             res_v, cen_sh, out_sem, stage_sem, *bufs_and_sems):
    lat_bufs = bufs_and_sems[:_NBUF]
    cen_bufs = bufs_and_sems[_NBUF:2 * _NBUF]
    fill_sems = bufs_and_sems[2 * _NBUF:3 * _NBUF]
    gat_sems = bufs_and_sems[3 * _NBUF:]
    sid = lax.axis_index("s")
    wid = sid * _NC + lax.axis_index("c")

    def fill(ch):
        b = ch % _NBUF
        row0 = wid * _RPW + ch * _CH
        return pltpu.async_copy(
            latent_hbm.at[pl.ds(row0, _CH)], lat_bufs[b], fill_sems[b])

    def gather(ch):
        b = ch % _NBUF
        return pltpu.async_copy(
            cen_sh.at[lab_v.at[pl.ds(ch * _CH, _CH)]], cen_bufs[b],
            gat_sems[b])

    fills = {ch: fill(ch) for ch in range(_NBUF)}

    base = jnp.minimum(sid * _CPT, _C - _CPT)
    lab_copy = pltpu.async_copy(
        labels_hbm.at[pl.ds(wid * _RPW, _RPW)], lab_v, stage_sem)
    stg_copy = pltpu.async_copy(
        centers_hbm.at[pl.ds(base, _CPT)], cen_sh.at[pl.ds(base, _CPT)],
        stage_sem)

    accs = tuple(jnp.zeros((_LANES,), jnp.float32) for _ in range(_JV))

    lab_copy.wait()
    stg_copy.wait()
    plsc.subcore_barrier()
    gats = {ch: gather(ch) for ch in range(_NBUF)}
    for ch in range(_NCH):
        b = ch % _NBUF
        fills.pop(ch).wait()
        gats.pop(ch).wait()
        lat_v, cen_v = lat_bufs[b], cen_bufs[b]

        @plsc.parallel_loop(0, _CH, 1, unroll=4, carry=accs)
        def row_loop(r, acc_in):
            new = []
            for j in range(_JV):
                lt = lat_v[r, pl.ds(j * _LANES, _LANES)]
                cn = cen_v[r, pl.ds(j * _LANES, _LANES)]
                d = lt - cn
                new.append(acc_in[j] + d * d)
            return tuple(new)

        accs = row_loop
        if ch + _NBUF < _NCH:
            fills[ch + _NBUF] = fill(ch + _NBUF)
            gats[ch + _NBUF] = gather(ch + _NBUF)

    total = accs[0]
    for j in range(1, _JV):
        total = total + accs[j]
    res_v[...] = total
    pltpu.async_copy(res_v, out_hbm.at[wid], out_sem).wait()


@jax.jit
def _center_loss_partials(latent, labels1d, centers):
    mesh = plsc.VectorSubcoreMesh(core_axis_name="c", subcore_axis_name="s")
    run = functools.partial(
        pl.kernel,
        out_type=jax.ShapeDtypeStruct((_NW, _LANES), jnp.float32),
        mesh=mesh,
        scratch_types=(
            [
                pltpu.VMEM((_RPW,), jnp.int32),
                pltpu.VMEM((_LANES,), jnp.float32),
                pltpu.VMEM_SHARED((_C, _D), jnp.float32),
                pltpu.SemaphoreType.DMA,
                pltpu.SemaphoreType.DMA,
            ]
            + [pltpu.VMEM((_CH, _D), jnp.float32) for _ in range(2 * _NBUF)]
            + [pltpu.SemaphoreType.DMA for _ in range(2 * _NBUF)]
        ),
    )(_sc_body)
    return run(latent, labels1d, centers)


def kernel(latent, labels, centers):
    partials = _center_loss_partials(latent, labels.astype(jnp.int32), centers)
    return jnp.sum(partials) / jnp.float32(_B)
